# baseline (speedup 1.0000x reference)
import jax
import jax.numpy as jnp
from jax import lax
from jax.experimental import pallas as pl
from jax.experimental.pallas import tpu as pltpu
from jax.experimental.pallas import tpu_sc as plsc

N = 10000
E = 320000
D = 128

NC = 2
NS = 16
NW = NC * NS

CHUNK = 256
CPW = 40
HALF = CPW // 2
E_PAD = NW * CPW * CHUNK
ACC_ROWS = 10240
ZROWS = ACC_ROWS // NS
OCHUNK = 80


def _sc_body(src_hbm, dst_hbm, h_hbm, partial_hbm,
             src_v, dst_v, rows_v, acc_sh, sem_a):
    c = lax.axis_index("c")
    s = lax.axis_index("s")
    wid = s * NC + c

    z16 = jnp.zeros((16,), jnp.float32)

    def zero_body(i, carry):
        r = i // (D // 16)
        col = (i % (D // 16)) * 16
        rows_v[r, pl.ds(col, 16)] = z16
        return carry

    lax.fori_loop(0, 128 * (D // 16), zero_body, 0)
    for k in range(ZROWS // 128):
        pltpu.sync_copy(rows_v.at[pl.ds(0, 128)], acc_sh.at[pl.ds(s * ZROWS + k * 128, 128)])
    plsc.subcore_barrier()

    base = wid * CPW
    pltpu.sync_copy(src_hbm.at[pl.ds(base * 256, CPW * 256)], src_v)
    for half in range(2):
        pltpu.sync_copy(dst_hbm.at[pl.ds(2 * base + half * CPW, CPW)], dst_v)
        for jj in range(HALF):
            j = half * HALF + jj
            pltpu.async_copy(h_hbm.at[src_v.at[pl.ds(j * 256, 256)]], rows_v, sem_a).wait()
            pltpu.sync_copy(rows_v.at[pl.ds(0, 128)],
                            acc_sh.at[dst_v.at[2 * jj]], add=True)
            pltpu.sync_copy(rows_v.at[pl.ds(128, 128)],
                            acc_sh.at[dst_v.at[2 * jj + 1]], add=True)
    plsc.subcore_barrier()

    def out_body(i, carry):
        off = (s + i * NS) * OCHUNK
        pltpu.sync_copy(acc_sh.at[pl.ds(off, OCHUNK)], rows_v.at[pl.ds(0, OCHUNK)])
        pltpu.sync_copy(rows_v.at[pl.ds(0, OCHUNK)], partial_hbm.at[c, pl.ds(off, OCHUNK)])
        return carry

    nunits = (N // OCHUNK - s + NS - 1) // NS
    lax.fori_loop(0, nunits, out_body, 0)


def _sc_segment_sum(src3d, dst3d, h):
    mesh = plsc.VectorSubcoreMesh(core_axis_name="c", subcore_axis_name="s")
    kern = pl.kernel(
        _sc_body,
        mesh=mesh,
        out_type=jax.ShapeDtypeStruct((NC, N, D), jnp.float32),
        scratch_types=[
            pltpu.VMEM((CPW * 256,), jnp.int32),
            pltpu.VMEM((CPW, 128), jnp.int32),
            pltpu.VMEM((256, D), jnp.float32),
            pltpu.VMEM_SHARED((ACC_ROWS, D), jnp.float32),
            pltpu.SemaphoreType.DMA,
        ],
    )
    return kern(src3d, dst3d, h)


def _tc_body(h_ref, p0_ref, p1_ref, ws_ref, wn_ref, b_ref, o_ref):
    dn = (((1,), (1,)), ((), ()))
    o_ref[...] = (
        lax.dot_general(h_ref[...], ws_ref[...], dn,
                        preferred_element_type=jnp.float32)
        + lax.dot_general(p0_ref[...] + p1_ref[...], wn_ref[...], dn,
                          preferred_element_type=jnp.float32)
        + b_ref[...]
    )


def _tc_combine(h, p0, p1, W_self, W_neigh, bsum):
    BR = 1000
    return pl.pallas_call(
        _tc_body,
        grid=(N // BR,),
        in_specs=[
            pl.BlockSpec((BR, D), lambda i: (i, 0)),
            pl.BlockSpec((BR, D), lambda i: (i, 0)),
            pl.BlockSpec((BR, D), lambda i: (i, 0)),
            pl.BlockSpec((D, D), lambda i: (0, 0)),
            pl.BlockSpec((D, D), lambda i: (0, 0)),
            pl.BlockSpec((1, D), lambda i: (0, 0)),
        ],
        out_specs=pl.BlockSpec((BR, D), lambda i: (i, 0)),
        out_shape=jax.ShapeDtypeStruct((N, D), jnp.float32),
    )(h, p0, p1, W_self, W_neigh, bsum)


def kernel(edge_index, h, W_self, b_self, W_neigh, b_neigh):
    pad = E_PAD - E
    src = jnp.concatenate([edge_index[0], jnp.zeros((pad,), jnp.int32)])
    pad_dst = N + (jnp.arange(pad, dtype=jnp.int32) % (ACC_ROWS - N))
    dst = jnp.concatenate([edge_index[1], pad_dst])
    src3d = src
    dst3d = dst.reshape(E_PAD // CHUNK * 2, 128)
    p = _sc_segment_sum(src3d, dst3d, h)
    bsum = (b_self + b_neigh).reshape(1, D)
    return _tc_combine(h, p[0], p[1], W_self, W_neigh, bsum)


# speedup vs baseline: 1.2701x; 1.2701x over previous
import jax
import jax.numpy as jnp
from jax import lax
from jax.experimental import pallas as pl
from jax.experimental.pallas import tpu as pltpu
from jax.experimental.pallas import tpu_sc as plsc

N = 10000
E = 320000
D = 128

NC = 2
NS = 16
NW = NC * NS

CHUNK = 128
CPW = 80
HALF = CPW // 2
E_PAD = NW * CPW * CHUNK
ACC_ROWS = 10240
ZROWS = ACC_ROWS // NS
OCHUNK = 80


def _sc_body(src_hbm, dst_hbm, h_hbm, partial_hbm,
             src_v, dst_v, rows_a, rows_b, acc_sh, sem_a, sem_b):
    c = lax.axis_index("c")
    s = lax.axis_index("s")
    wid = s * NC + c

    bufs = (rows_a, rows_b)
    sems = (sem_a, sem_b)

    def start(j):
        b = j % 2
        return pltpu.async_copy(h_hbm.at[src_v.at[j]], bufs[b], sems[b])

    pltpu.sync_copy(src_hbm.at[pl.ds(wid * CPW, CPW)], src_v)
    desc = [None] * CPW
    desc[0] = start(0)

    pltpu.sync_copy(h_hbm.at[pl.ds(N, CHUNK)], rows_b)
    for k in range(ZROWS // CHUNK):
        pltpu.sync_copy(rows_b, acc_sh.at[pl.ds(s * ZROWS + k * CHUNK, CHUNK)])
    plsc.subcore_barrier()

    for half in range(2):
        pltpu.sync_copy(
            dst_hbm.at[pl.ds(wid * CPW + half * HALF, HALF)], dst_v)
        for jj in range(HALF):
            j = half * HALF + jj
            desc[j].wait()
            if j + 1 < CPW:
                desc[j + 1] = start(j + 1)
            pltpu.sync_copy(bufs[j % 2], acc_sh.at[dst_v.at[jj]], add=True)
    plsc.subcore_barrier()

    nunits = (N // OCHUNK - s + NS - 1) // NS

    def out_body(i, carry):
        off = (s + i * NS) * OCHUNK
        pltpu.sync_copy(acc_sh.at[pl.ds(off, OCHUNK)],
                        rows_a.at[pl.ds(0, OCHUNK)])
        pltpu.sync_copy(rows_a.at[pl.ds(0, OCHUNK)],
                        partial_hbm.at[c, pl.ds(off, OCHUNK)])
        return carry

    lax.fori_loop(0, nunits, out_body, 0)


def _sc_segment_sum(src2d, dst2d, h_pad):
    mesh = plsc.VectorSubcoreMesh(core_axis_name="c", subcore_axis_name="s")
    kern = pl.kernel(
        _sc_body,
        mesh=mesh,
        out_type=jax.ShapeDtypeStruct((NC, N, D), jnp.float32),
        scratch_types=[
            pltpu.VMEM((CPW, CHUNK), jnp.int32),
            pltpu.VMEM((HALF, CHUNK), jnp.int32),
            pltpu.VMEM((CHUNK, D), jnp.float32),
            pltpu.VMEM((CHUNK, D), jnp.float32),
            pltpu.VMEM_SHARED((ACC_ROWS, D), jnp.float32),
            pltpu.SemaphoreType.DMA,
            pltpu.SemaphoreType.DMA,
        ],
    )
    return kern(src2d, dst2d, h_pad)


def _tc_body(h_ref, p0_ref, p1_ref, ws_ref, wn_ref, b_ref, o_ref):
    dn = (((1,), (1,)), ((), ()))
    o_ref[...] = (
        lax.dot_general(h_ref[...], ws_ref[...], dn,
                        preferred_element_type=jnp.float32)
        + lax.dot_general(p0_ref[...] + p1_ref[...], wn_ref[...], dn,
                          preferred_element_type=jnp.float32)
        + b_ref[...]
    )


def _tc_combine(h, p0, p1, W_self, W_neigh, bsum):
    BR = 1000
    return pl.pallas_call(
        _tc_body,
        grid=(N // BR,),
        in_specs=[
            pl.BlockSpec((BR, D), lambda i: (i, 0)),
            pl.BlockSpec((BR, D), lambda i: (i, 0)),
            pl.BlockSpec((BR, D), lambda i: (i, 0)),
            pl.BlockSpec((D, D), lambda i: (0, 0)),
            pl.BlockSpec((D, D), lambda i: (0, 0)),
            pl.BlockSpec((1, D), lambda i: (0, 0)),
        ],
        out_specs=pl.BlockSpec((BR, D), lambda i: (i, 0)),
        out_shape=jax.ShapeDtypeStruct((N, D), jnp.float32),
    )(h, p0, p1, W_self, W_neigh, bsum)


def kernel(edge_index, h, W_self, b_self, W_neigh, b_neigh):
    pad = E_PAD - E
    src = jnp.concatenate([edge_index[0], jnp.zeros((pad,), jnp.int32)])
    pad_dst = N + (jnp.arange(pad, dtype=jnp.int32) % (ACC_ROWS - N))
    dst = jnp.concatenate([edge_index[1], pad_dst])
    src2d = src.reshape(E_PAD // CHUNK, CHUNK)
    dst2d = dst.reshape(E_PAD // CHUNK, CHUNK)
    h_pad = jnp.concatenate([h, jnp.zeros((CHUNK, D), jnp.float32)], axis=0)
    p = _sc_segment_sum(src2d, dst2d, h_pad)
    bsum = (b_self + b_neigh).reshape(1, D)
    return _tc_combine(h, p[0], p[1], W_self, W_neigh, bsum)
